# baseline (speedup 1.0000x reference)
import functools

import jax
import jax.numpy as jnp
from jax import lax
from jax.experimental import pallas as pl
from jax.experimental.pallas import tpu as pltpu
from jax.experimental.pallas import tpu_sc as plsc

BATCH = 16384
HIST = 50
DIM = 64
VOCAB1 = 1000001
NC = 2
NS = 16
NW = NC * NS
PR = 500000
NFULL = 7812
FPW = NFULL // NW

BW = BATCH // NW
CB = 256
NSUB = BW // CB

_mesh = plsc.VectorSubcoreMesh(core_axis_name="c", subcore_axis_name="s")


@functools.partial(
    pl.kernel,
    out_type=jax.ShapeDtypeStruct((PR, 128), jnp.float32),
    mesh=_mesh,
    scratch_types=[
        pltpu.VMEM((8, 8, 128), jnp.float32),
        pltpu.VMEM((8, 8, 128), jnp.float32),
        pltpu.VMEM((64, 128), jnp.float32),
        pltpu.VMEM((64, 128), jnp.float32),
        pltpu.SemaphoreType.DMA,
        pltpu.SemaphoreType.DMA,
        pltpu.SemaphoreType.DMA,
        pltpu.SemaphoreType.DMA,
    ],
    compiler_params=pltpu.CompilerParams(
        use_tc_tiling_on_sc=True, needs_layout_passes=False
    ),
)
def _transpose_kernel(tt_hbm, out_hbm, slab0, slab1, to0, to1,
                      si0, si1, so0, so1):
    wid = lax.axis_index("s") * NC + lax.axis_index("c")
    slab = (slab0, slab1)
    to = (to0, to1)
    s_i = (si0, si1)
    s_o = (so0, so1)
    iota = lax.iota(jnp.int32, 16)
    t_hi = (iota & 8) >> 3
    r_ids = iota & 7
    diag = [(iota + j) & 15 for j in range(16)]

    def fire_in(cb, b, ncols):
        for t in range(8):
            pltpu.async_copy(
                tt_hbm.at[pl.ds(8 * t, 8), pl.ds(cb * 128, ncols)],
                slab[b].at[t, :, pl.ds(0, ncols)],
                s_i[b],
            )

    def wait_in(cb, b, ncols):
        for t in range(8):
            pltpu.make_async_copy(
                tt_hbm.at[pl.ds(8 * t, 8), pl.ds(cb * 128, ncols)],
                slab[b].at[t, :, pl.ds(0, ncols)],
                s_i[b],
            ).wait()

    def out_dst(cb, b, nrows):
        return out_hbm.at[pl.ds(cb * 64, nrows)]

    def transpose_block(b, nv):
        def qbody(q, carry):
            t_ids = 2 * q + t_hi
            f_ids = 16 * q + iota
            for v0 in range(0, nv, 16):
                for j in range(16):
                    v_ids = v0 + diag[j]
                    v = plsc.load_gather(slab[b], [t_ids, r_ids, v_ids])
                    plsc.store_scatter(
                        to[b],
                        [v_ids >> 1, ((v_ids & 1) << 6) + f_ids],
                        v,
                    )
            return carry
        lax.fori_loop(0, 4, qbody, 0)

    fire_in(wid, 0, 128)

    def body(k, carry):
        for b in range(2):
            i = 2 * k + b
            cb = wid + 32 * i

            if b == 0:
                fire_in(wid + 32 * (i + 1), 1, 128)
            else:
                @pl.when(k < FPW // 2 - 1)
                def _fire():
                    fire_in(wid + 32 * (i + 1), 0, 128)

            wait_in(cb, b, 128)

            @pl.when(k > 0)
            def _wait_out():
                pltpu.make_async_copy(
                    to[b], out_dst(cb - 64, b, 64), s_o[b]
                ).wait()

            transpose_block(b, 128)
            pltpu.async_copy(to[b], out_dst(cb, b, 64), s_o[b])
        return carry

    lax.fori_loop(0, FPW // 2, body, 0)

    pltpu.make_async_copy(to0, out_dst(wid + 32 * (FPW - 2), 0, 64), so0).wait()
    pltpu.make_async_copy(to1, out_dst(wid + 32 * (FPW - 1), 1, 64), so1).wait()

    @pl.when(wid <= 3)
    def _tail_full():
        cb = NFULL - 4 + wid
        fire_in(cb, 0, 128)
        wait_in(cb, 0, 128)
        transpose_block(0, 128)
        pltpu.async_copy(to0, out_dst(cb, 0, 64), so0)
        pltpu.make_async_copy(to0, out_dst(cb, 0, 64), so0).wait()

    @pl.when(wid == 4)
    def _tail_half():
        cb = NFULL
        fire_in(cb, 0, 64)
        wait_in(cb, 0, 64)
        transpose_block(0, 64)
        pltpu.async_copy(to0.at[pl.ds(0, 32)], out_dst(cb, 0, 32), so0)
        pltpu.make_async_copy(to0.at[pl.ds(0, 32)], out_dst(cb, 0, 32), so0).wait()


@functools.partial(
    pl.kernel,
    out_type=jax.ShapeDtypeStruct((HIST, DIM, BATCH), jnp.float32),
    mesh=_mesh,
    scratch_types=[
        pltpu.VMEM((HIST, BW), jnp.int32),
        pltpu.VMEM((CB,), jnp.int32),
        pltpu.VMEM((CB,), jnp.int32),
        pltpu.VMEM((CB, 128), jnp.float32),
        pltpu.VMEM((CB, 128), jnp.float32),
        pltpu.VMEM((DIM, CB), jnp.float32),
        pltpu.VMEM((DIM, CB), jnp.float32),
        pltpu.SemaphoreType.DMA,
        pltpu.SemaphoreType.DMA,
        pltpu.SemaphoreType.DMA,
        pltpu.SemaphoreType.DMA,
        pltpu.SemaphoreType.DMA,
    ],
    compiler_params=pltpu.CompilerParams(
        use_tc_tiling_on_sc=False, needs_layout_passes=False
    ),
)
def _gather_kernel(table_hbm, idxt_hbm, out_hbm,
                   idx_all, ip0, ip1, rows0, rows1, rt0, rt1,
                   s_idx, sg0, sg1, sw0, sw1):
    wid = lax.axis_index("s") * NC + lax.axis_index("c")
    boff = wid * BW
    ip = (ip0, ip1)
    rows = (rows0, rows1)
    rt = (rt0, rt1)
    s_g = (sg0, sg1)
    s_w = (sw0, sw1)
    iota = lax.iota(jnp.int32, 16)
    diag = [(iota + j) & 15 for j in range(16)]

    def fire_gather(h, sub, b):
        for g in range(CB // 16):
            v = idx_all[h, pl.ds(sub * CB + g * 16, 16)]
            ip[b][pl.ds(g * 16, 16)] = v >> 1
        half = CB // 2
        for p in range(2):
            pltpu.async_copy(
                table_hbm.at[ip[b].at[pl.ds(p * half, half)]],
                rows[b].at[pl.ds(p * half, half)],
                s_g[b],
            )

    def wait_gather(b):
        half = CB // 2
        for p in range(2):
            pltpu.make_async_copy(
                table_hbm.at[ip[b].at[pl.ds(p * half, half)]],
                rows[b].at[pl.ds(p * half, half)],
                s_g[b],
            ).wait()

    def transpose(h, sub, b):
        def rblk(r, carry):
            r0 = r * 16
            rids = r0 + iota
            idx_vals = idx_all[h, pl.ds(sub * CB + r0, 16)]
            halfsel = (idx_vals & 1) << 6
            for c0 in (0, 16, 32, 48):
                cid_list = [c0 + diag[j] for j in range(16)]
                vs = [
                    plsc.load_gather(rows[b], [rids, cids + halfsel])
                    for cids in cid_list
                ]
                for cids, v in zip(cid_list, vs):
                    plsc.store_scatter(rt[b], [cids, rids], v)
            return carry
        lax.fori_loop(0, CB // 16, rblk, 0)

    pltpu.async_copy(idxt_hbm.at[:, pl.ds(boff, BW)], idx_all, s_idx).wait()
    fire_gather(0, 0, 0)

    def body(h, carry):
        for b in range(2):
            if b == 0:
                fire_gather(h, 1, 1)
            else:
                @pl.when(h < HIST - 1)
                def _fire():
                    fire_gather(h + 1, 0, 0)

            wait_gather(b)

            @pl.when(h > 0)
            def _wait_write():
                pltpu.make_async_copy(
                    rt[b], out_hbm.at[h, :, pl.ds(boff + b * CB, CB)], s_w[b]
                ).wait()

            transpose(h, b, b)
            pltpu.async_copy(
                rt[b], out_hbm.at[h, :, pl.ds(boff + b * CB, CB)], s_w[b]
            )
        return carry

    lax.fori_loop(0, HIST, body, 0)

    h_last = HIST - 1
    pltpu.make_async_copy(
        rt0, out_hbm.at[h_last, :, pl.ds(boff, CB)], sw0
    ).wait()
    pltpu.make_async_copy(
        rt1, out_hbm.at[h_last, :, pl.ds(boff + CB, CB)], sw1
    ).wait()


def kernel(ivectors, data):
    tt = ivectors.T
    tdense = _transpose_kernel(tt)
    idx_t = data.T.astype(jnp.int32)
    out_p = _gather_kernel(tdense, idx_t)
    return out_p.transpose(2, 0, 1)


# speedup vs baseline: 1.4830x; 1.4830x over previous
import functools

import jax
import jax.numpy as jnp
from jax import lax
from jax.experimental import pallas as pl
from jax.experimental.pallas import tpu as pltpu
from jax.experimental.pallas import tpu_sc as plsc

BATCH = 16384
HIST = 50
DIM = 64
VOCAB1 = 1000001
NC = 2
NS = 16
NW = NC * NS
PR = 500000
NFULL = 7812
FPW = NFULL // NW

BW = BATCH // NW
CB = 256
NSUB = BW // CB

_mesh = plsc.VectorSubcoreMesh(core_axis_name="c", subcore_axis_name="s")


@functools.partial(
    pl.kernel,
    out_type=jax.ShapeDtypeStruct((PR, 128), jnp.float32),
    mesh=_mesh,
    scratch_types=[
        pltpu.VMEM((8, 8, 128), jnp.float32),
        pltpu.VMEM((8, 8, 128), jnp.float32),
        pltpu.VMEM((64, 128), jnp.float32),
        pltpu.VMEM((64, 128), jnp.float32),
        pltpu.SemaphoreType.DMA,
        pltpu.SemaphoreType.DMA,
        pltpu.SemaphoreType.DMA,
        pltpu.SemaphoreType.DMA,
    ],
    compiler_params=pltpu.CompilerParams(
        use_tc_tiling_on_sc=True, needs_layout_passes=False
    ),
)
def _transpose_kernel(tt_hbm, out_hbm, slab0, slab1, to0, to1,
                      si0, si1, so0, so1):
    wid = lax.axis_index("s") * NC + lax.axis_index("c")
    slab = (slab0, slab1)
    to = (to0, to1)
    s_i = (si0, si1)
    s_o = (so0, so1)
    iota = lax.iota(jnp.int32, 16)
    t_hi = (iota & 8) >> 3
    r_ids = iota & 7
    diag = [(iota + j) & 15 for j in range(16)]

    def fire_in(cb, b, ncols):
        for t in range(8):
            pltpu.async_copy(
                tt_hbm.at[pl.ds(8 * t, 8), pl.ds(cb * 128, ncols)],
                slab[b].at[t, :, pl.ds(0, ncols)],
                s_i[b],
            )

    def wait_in(cb, b, ncols):
        for t in range(8):
            pltpu.make_async_copy(
                tt_hbm.at[pl.ds(8 * t, 8), pl.ds(cb * 128, ncols)],
                slab[b].at[t, :, pl.ds(0, ncols)],
                s_i[b],
            ).wait()

    def out_dst(cb, b, nrows):
        return out_hbm.at[pl.ds(cb * 64, nrows)]

    def transpose_block(b, nv):
        def qbody(q, carry):
            t_ids = 2 * q + t_hi
            f_ids = 16 * q + iota
            for v0 in range(0, nv, 16):
                v_list = [v0 + diag[j] for j in range(16)]
                vs = [
                    plsc.load_gather(slab[b], [t_ids, r_ids, v_ids])
                    for v_ids in v_list
                ]
                for v_ids, v in zip(v_list, vs):
                    plsc.store_scatter(
                        to[b],
                        [v_ids >> 1, ((v_ids & 1) << 6) + f_ids],
                        v,
                    )
            return carry
        lax.fori_loop(0, 4, qbody, 0)

    fire_in(wid, 0, 128)

    def body(k, carry):
        for b in range(2):
            i = 2 * k + b
            cb = wid + 32 * i

            if b == 0:
                fire_in(wid + 32 * (i + 1), 1, 128)
            else:
                @pl.when(k < FPW // 2 - 1)
                def _fire():
                    fire_in(wid + 32 * (i + 1), 0, 128)

            wait_in(cb, b, 128)

            @pl.when(k > 0)
            def _wait_out():
                pltpu.make_async_copy(
                    to[b], out_dst(cb - 64, b, 64), s_o[b]
                ).wait()

            transpose_block(b, 128)
            pltpu.async_copy(to[b], out_dst(cb, b, 64), s_o[b])
        return carry

    lax.fori_loop(0, FPW // 2, body, 0)

    pltpu.make_async_copy(to0, out_dst(wid + 32 * (FPW - 2), 0, 64), so0).wait()
    pltpu.make_async_copy(to1, out_dst(wid + 32 * (FPW - 1), 1, 64), so1).wait()

    @pl.when(wid <= 3)
    def _tail_full():
        cb = NFULL - 4 + wid
        fire_in(cb, 0, 128)
        wait_in(cb, 0, 128)
        transpose_block(0, 128)
        pltpu.async_copy(to0, out_dst(cb, 0, 64), so0)
        pltpu.make_async_copy(to0, out_dst(cb, 0, 64), so0).wait()

    @pl.when(wid == 4)
    def _tail_half():
        cb = NFULL
        fire_in(cb, 0, 64)
        wait_in(cb, 0, 64)
        transpose_block(0, 64)
        pltpu.async_copy(to0.at[pl.ds(0, 32)], out_dst(cb, 0, 32), so0)
        pltpu.make_async_copy(to0.at[pl.ds(0, 32)], out_dst(cb, 0, 32), so0).wait()


@functools.partial(
    pl.kernel,
    out_type=jax.ShapeDtypeStruct((HIST, DIM, BATCH), jnp.float32),
    mesh=_mesh,
    scratch_types=[
        pltpu.VMEM((HIST, BW), jnp.int32),
        pltpu.VMEM((CB,), jnp.int32),
        pltpu.VMEM((CB,), jnp.int32),
        pltpu.VMEM((CB, 128), jnp.float32),
        pltpu.VMEM((CB, 128), jnp.float32),
        pltpu.VMEM((DIM, CB), jnp.float32),
        pltpu.VMEM((DIM, CB), jnp.float32),
        pltpu.SemaphoreType.DMA,
        pltpu.SemaphoreType.DMA,
        pltpu.SemaphoreType.DMA,
        pltpu.SemaphoreType.DMA,
        pltpu.SemaphoreType.DMA,
    ],
    compiler_params=pltpu.CompilerParams(
        use_tc_tiling_on_sc=False, needs_layout_passes=False
    ),
)
def _gather_kernel(table_hbm, idxt_hbm, out_hbm,
                   idx_all, ip0, ip1, rows0, rows1, rt0, rt1,
                   s_idx, sg0, sg1, sw0, sw1):
    wid = lax.axis_index("s") * NC + lax.axis_index("c")
    boff = wid * BW
    ip = (ip0, ip1)
    rows = (rows0, rows1)
    rt = (rt0, rt1)
    s_g = (sg0, sg1)
    s_w = (sw0, sw1)
    iota = lax.iota(jnp.int32, 16)
    diag = [(iota + j) & 15 for j in range(16)]

    def fire_gather(h, sub, b):
        for g in range(CB // 16):
            v = idx_all[h, pl.ds(sub * CB + g * 16, 16)]
            ip[b][pl.ds(g * 16, 16)] = v >> 1
        half = CB // 2
        for p in range(2):
            pltpu.async_copy(
                table_hbm.at[ip[b].at[pl.ds(p * half, half)]],
                rows[b].at[pl.ds(p * half, half)],
                s_g[b],
            )

    def wait_gather(b):
        half = CB // 2
        for p in range(2):
            pltpu.make_async_copy(
                table_hbm.at[ip[b].at[pl.ds(p * half, half)]],
                rows[b].at[pl.ds(p * half, half)],
                s_g[b],
            ).wait()

    def transpose(h, sub, b):
        def rblk(r, carry):
            r0 = r * 16
            rids = r0 + iota
            idx_vals = idx_all[h, pl.ds(sub * CB + r0, 16)]
            halfsel = (idx_vals & 1) << 6
            for c0 in (0, 16, 32, 48):
                cid_list = [c0 + diag[j] for j in range(16)]
                vs = [
                    plsc.load_gather(rows[b], [rids, cids + halfsel])
                    for cids in cid_list
                ]
                for cids, v in zip(cid_list, vs):
                    plsc.store_scatter(rt[b], [cids, rids], v)
            return carry
        lax.fori_loop(0, CB // 16, rblk, 0)

    pltpu.async_copy(idxt_hbm.at[:, pl.ds(boff, BW)], idx_all, s_idx).wait()
    fire_gather(0, 0, 0)

    def body(h, carry):
        for b in range(2):
            if b == 0:
                fire_gather(h, 1, 1)
            else:
                @pl.when(h < HIST - 1)
                def _fire():
                    fire_gather(h + 1, 0, 0)

            wait_gather(b)

            @pl.when(h > 0)
            def _wait_write():
                pltpu.make_async_copy(
                    rt[b], out_hbm.at[h, :, pl.ds(boff + b * CB, CB)], s_w[b]
                ).wait()

            transpose(h, b, b)
            pltpu.async_copy(
                rt[b], out_hbm.at[h, :, pl.ds(boff + b * CB, CB)], s_w[b]
            )
        return carry

    lax.fori_loop(0, HIST, body, 0)

    h_last = HIST - 1
    pltpu.make_async_copy(
        rt0, out_hbm.at[h_last, :, pl.ds(boff, CB)], sw0
    ).wait()
    pltpu.make_async_copy(
        rt1, out_hbm.at[h_last, :, pl.ds(boff + CB, CB)], sw1
    ).wait()


def kernel(ivectors, data):
    tt = ivectors.T
    tdense = _transpose_kernel(tt)
    idx_t = data.T.astype(jnp.int32)
    out_p = _gather_kernel(tdense, idx_t)
    return out_p.transpose(2, 0, 1)
